# Initial kernel scaffold; baseline (speedup 1.0000x reference)
import jax
import jax.numpy as jnp
from jax.experimental import pallas as pl


def kernel(hidden_states, top_k_index, top_k_weights, gate_up_proj, down_proj, per_expert_scale):
    raise NotImplementedError("write your pallas kernel here")


# speedup vs baseline: 1.2185x; 1.2185x over previous
import functools

import jax
import jax.numpy as jnp
from jax.experimental import pallas as pl
from jax.experimental.pallas import tpu as pltpu

_NUM_EXPERTS = 8
_INTER = 4096
_HIDDEN = 2048
_TOKENS = 2048
_TOP_K = 2

_TB = 1024
_IB = 256


def _dense_body(idx_ref, w_ref, scale_ref, x_ref, g_ref, u_ref, d_ref, out_ref):
    e = pl.program_id(1)
    j = pl.program_id(2)

    @pl.when((e == 0) & (j == 0))
    def _():
        out_ref[...] = jnp.zeros_like(out_ref)

    x = x_ref[...]
    g = jax.lax.dot_general(x, g_ref[0], (((1,), (1,)), ((), ())),
                            preferred_element_type=jnp.float32)
    u = jax.lax.dot_general(x, u_ref[0], (((1,), (1,)), ((), ())),
                            preferred_element_type=jnp.float32)
    h = g * jax.lax.logistic(g) * u
    part = jax.lax.dot_general(h, d_ref[0], (((1,), (1,)), ((), ())),
                               preferred_element_type=jnp.float32)
    mask = (idx_ref[...] == e).astype(jnp.float32)
    w = jnp.sum(w_ref[...] * mask, axis=1) * scale_ref[e]
    out_ref[...] += part * w[:, None]


def kernel(hidden_states, top_k_index, top_k_weights, gate_up_proj, down_proj,
           per_expert_scale):
    t_blocks = _TOKENS // _TB
    j_blocks = _INTER // _IB
    grid = (t_blocks, _NUM_EXPERTS, j_blocks)
    out = pl.pallas_call(
        _dense_body,
        grid=grid,
        in_specs=[
            pl.BlockSpec((_TB, _TOP_K), lambda t, e, j: (t, 0)),
            pl.BlockSpec((_TB, _TOP_K), lambda t, e, j: (t, 0)),
            pl.BlockSpec(memory_space=pltpu.SMEM),
            pl.BlockSpec((_TB, _HIDDEN), lambda t, e, j: (t, 0)),
            pl.BlockSpec((1, _IB, _HIDDEN), lambda t, e, j: (e, j, 0)),
            pl.BlockSpec((1, _IB, _HIDDEN), lambda t, e, j: (e, j_blocks + j, 0)),
            pl.BlockSpec((1, _HIDDEN, _IB), lambda t, e, j: (e, 0, j)),
        ],
        out_specs=pl.BlockSpec((_TB, _HIDDEN), lambda t, e, j: (t, 0)),
        out_shape=jax.ShapeDtypeStruct((_TOKENS, _HIDDEN), jnp.float32),
    )(top_k_index.astype(jnp.int32), top_k_weights, per_expert_scale,
      hidden_states, gate_up_proj, gate_up_proj, down_proj)
    return out


# speedup vs baseline: 1.2884x; 1.0574x over previous
import functools

import jax
import jax.numpy as jnp
from jax.experimental import pallas as pl
from jax.experimental.pallas import tpu as pltpu

_NUM_EXPERTS = 8
_INTER = 4096
_HIDDEN = 2048
_TOKENS = 2048
_TOP_K = 2

_TB = 2048
_IB = 256


def _dense_body(idx_ref, w_ref, scale_ref, x_ref, g_ref, u_ref, d_ref, out_ref):
    e = pl.program_id(1)
    j = pl.program_id(2)

    @pl.when((e == 0) & (j == 0))
    def _():
        out_ref[...] = jnp.zeros_like(out_ref)

    x = x_ref[...]
    g = jax.lax.dot_general(x, g_ref[0], (((1,), (1,)), ((), ())),
                            preferred_element_type=jnp.float32)
    u = jax.lax.dot_general(x, u_ref[0], (((1,), (1,)), ((), ())),
                            preferred_element_type=jnp.float32)
    h = g * jax.lax.logistic(g) * u
    part = jax.lax.dot_general(h, d_ref[0], (((1,), (1,)), ((), ())),
                               preferred_element_type=jnp.float32)
    mask = (idx_ref[...] == e).astype(jnp.float32)
    w = jnp.sum(w_ref[...] * mask, axis=1) * scale_ref[e]
    out_ref[...] += part * w[:, None]


def kernel(hidden_states, top_k_index, top_k_weights, gate_up_proj, down_proj,
           per_expert_scale):
    t_blocks = _TOKENS // _TB
    j_blocks = _INTER // _IB
    grid = (t_blocks, _NUM_EXPERTS, j_blocks)
    out = pl.pallas_call(
        _dense_body,
        grid=grid,
        in_specs=[
            pl.BlockSpec((_TB, _TOP_K), lambda t, e, j: (t, 0)),
            pl.BlockSpec((_TB, _TOP_K), lambda t, e, j: (t, 0)),
            pl.BlockSpec(memory_space=pltpu.SMEM),
            pl.BlockSpec((_TB, _HIDDEN), lambda t, e, j: (t, 0)),
            pl.BlockSpec((1, _IB, _HIDDEN), lambda t, e, j: (e, j, 0)),
            pl.BlockSpec((1, _IB, _HIDDEN), lambda t, e, j: (e, j_blocks + j, 0)),
            pl.BlockSpec((1, _HIDDEN, _IB), lambda t, e, j: (e, 0, j)),
        ],
        out_specs=pl.BlockSpec((_TB, _HIDDEN), lambda t, e, j: (t, 0)),
        out_shape=jax.ShapeDtypeStruct((_TOKENS, _HIDDEN), jnp.float32),
    )(top_k_index.astype(jnp.int32), top_k_weights, per_expert_scale,
      hidden_states, gate_up_proj, gate_up_proj, down_proj)
    return out
